# baseline (speedup 1.0000x reference)
import functools

import jax
import jax.numpy as jnp
from jax import lax
from jax.experimental import pallas as pl
from jax.experimental.pallas import tpu as pltpu
from jax.experimental.pallas import tpu_sc as plsc

E = 320000
D = 128
G = 256
NC = 2
NS = 16
L = 16
NW = NC * NS
EW = E // NW
C = 80
NCHUNK = EW // C
GROUPS = C // L
NJ = D // L


def _tanh(z):
    a = jnp.abs(z)
    e = jnp.exp(a * (-2.0))
    return jnp.sign(z) * (1.0 - e) / (1.0 + e)


def _sc_body(feats_hbm, ids_hbm, w_hbm, b_hbm, out_hbm,
             buf2, ids2, acc, wbuf, bbuf, smat, semA, semB):
    wid = lax.axis_index("s") * NC + lax.axis_index("c")
    row0 = wid * EW

    pltpu.sync_copy(w_hbm, wbuf)
    pltpu.sync_copy(b_hbm, bbuf)
    bv = bbuf[...]
    wv = [wbuf[pl.ds(j * L, L)] for j in range(NJ)]
    lanes = lax.iota(jnp.int32, L)

    def _zero(i, _):
        acc[pl.ds(i * L, L)] = jnp.zeros((L,), jnp.float32)
        return 0
    lax.fori_loop(0, G * D // L, _zero, 0)

    sems = (semA, semB)

    def _start(ci, slot):
        base = row0 + ci * C
        pltpu.async_copy(feats_hbm.at[pl.ds(base, C)], buf2.at[slot],
                         sems[slot])
        pltpu.async_copy(ids_hbm.at[pl.ds(base, C)], ids2.at[slot],
                         sems[slot])

    def _wait(slot):
        pltpu.make_async_copy(feats_hbm.at[pl.ds(0, C)], buf2.at[slot],
                              sems[slot]).wait()
        pltpu.make_async_copy(ids_hbm.at[pl.ds(0, C)], ids2.at[slot],
                              sems[slot]).wait()

    def _dg(x, perm):
        return x.at[perm].get(mode="promise_in_bounds")

    splat0 = jnp.zeros((L,), jnp.int32)
    splat1 = jnp.full((L,), 1, jnp.int32)
    p1, p2, p4, p8 = (lanes ^ 1, lanes ^ 2, lanes ^ 4, lanes ^ 8)
    m1 = (lanes & 1) == 0

    def _pair_accumulate(slot, row_a, gsum):
        xa = [buf2[slot, row_a, pl.ds(j * L, L)] for j in range(NJ)]
        xb = [buf2[slot, row_a + 1, pl.ds(j * L, L)] for j in range(NJ)]
        sa = xa[0] * wv[0]
        sb = xb[0] * wv[0]
        for j in range(1, NJ):
            sa = sa + xa[j] * wv[j]
            sb = sb + xb[j] * wv[j]
        c = jnp.where(m1, sa, sb) + jnp.where(m1, _dg(sa, p1), _dg(sb, p1))
        t = c + _dg(c, p2)
        t = t + _dg(t, p4)
        t = t + _dg(t, p8)
        wp = _tanh(t + bv)
        wa = _dg(wp, splat0)
        wb = _dg(wp, splat1)
        return [gsum[j] + xa[j] * wa + xb[j] * wb for j in range(NJ)]

    def _flush(seg, gsum):
        off = seg * D
        for j in range(NJ):
            plsc.addupdate(acc.at[pl.ds(off + j * L, L)], gsum[j])

    def _process(slot):
        id_first = jnp.minimum(ids2[slot, pl.ds(0, L)], G - 1)[0]
        id_last = jnp.minimum(ids2[slot, pl.ds(C - L, L)], G - 1)[L - 1]

        def _chunk_uniform(_):
            def _grp(gi, gsum):
                r0 = gi * L
                for r in range(0, L, 2):
                    gsum = _pair_accumulate(slot, r0 + r, gsum)
                return gsum
            gsum = lax.fori_loop(
                0, GROUPS, _grp, [jnp.zeros((L,), jnp.float32)] * NJ)
            _flush(id_first, gsum)
            return 0

        def _chunk_mixed(_):
            def _group(gi, _):
                r0 = gi * L
                idv = jnp.minimum(ids2[slot, pl.ds(r0, L)], G - 1)
                seg_first = idv[0]
                seg_last = idv[L - 1]

                def _uniform(_):
                    gsum = [jnp.zeros((L,), jnp.float32) for _ in range(NJ)]
                    for r in range(0, L, 2):
                        gsum = _pair_accumulate(slot, r0 + r, gsum)
                    _flush(seg_first, gsum)
                    return 0

                def _mixed(_):
                    for r in range(L):
                        row = r0 + r
                        s = buf2[slot, row, pl.ds(0, L)] * wv[0]
                        for j in range(1, NJ):
                            s = s + buf2[slot, row, pl.ds(j * L, L)] * wv[j]
                        smat[pl.ds(r * L, L)] = s
                    hv = plsc.load_gather(smat, [lanes * L])
                    for j in range(1, L):
                        hv = hv + plsc.load_gather(smat, [lanes * L + j])
                    wg = _tanh(hv + bv)
                    for r in range(L):
                        rfull = jnp.full((L,), r, jnp.int32)
                        wsp = _dg(wg, rfull)
                        base_idx = _dg(idv, rfull) * D + lanes
                        for j in range(NJ):
                            x = buf2[slot, r0 + r, pl.ds(j * L, L)] * wsp
                            plsc.addupdate_scatter(acc, [base_idx + j * L], x)
                    return 0

                lax.cond(seg_first == seg_last, _uniform, _mixed, 0)
                return 0

            lax.fori_loop(0, GROUPS, _group, 0)
            return 0

        lax.cond(id_first == id_last, _chunk_uniform, _chunk_mixed, 0)

    _start(0, 0)

    def _chunk(ci, _):
        parity = ci % 2

        @pl.when(parity == 0)
        def _():
            _wait(0)
            pl.when(ci + 1 < NCHUNK)(lambda: _start(ci + 1, 1))
            _process(0)

        @pl.when(parity == 1)
        def _():
            _wait(1)
            pl.when(ci + 1 < NCHUNK)(lambda: _start(ci + 1, 0))
            _process(1)

        return 0

    lax.fori_loop(0, NCHUNK, _chunk, 0)
    pltpu.sync_copy(acc, out_hbm.at[wid])


def _combine_body(parts_ref, o_ref):
    o_ref[...] = jnp.sum(parts_ref[...], axis=0)


@jax.jit
def _run(edge_feats, ids32, w_flat, b_pad):
    mesh = plsc.VectorSubcoreMesh(core_axis_name="c", subcore_axis_name="s",
                                  num_cores=NC, num_subcores=NS)
    sc = pl.kernel(
        _sc_body,
        out_type=jax.ShapeDtypeStruct((NW, G * D), jnp.float32),
        mesh=mesh,
        compiler_params=pltpu.CompilerParams(needs_layout_passes=False),
        scratch_types=[
            pltpu.VMEM((2, C, D), jnp.float32),
            pltpu.VMEM((2, C), jnp.int32),
            pltpu.VMEM((G * D,), jnp.float32),
            pltpu.VMEM((D,), jnp.float32),
            pltpu.VMEM((L,), jnp.float32),
            pltpu.VMEM((L * L,), jnp.float32),
            pltpu.SemaphoreType.DMA,
            pltpu.SemaphoreType.DMA,
        ],
    )
    parts = sc(edge_feats, ids32, w_flat, b_pad)
    out = pl.pallas_call(
        _combine_body,
        out_shape=jax.ShapeDtypeStruct((G, D), jnp.float32),
    )(parts.reshape(NW, G, D))
    return out


def kernel(edge_feats, segment_ids, num_graphs, W, b):
    ids32 = segment_ids.astype(jnp.int32)
    w_flat = W.reshape(D)
    b_pad = jnp.broadcast_to(b.reshape(1), (L,)).astype(jnp.float32)
    return _run(edge_feats, ids32, w_flat, b_pad)


# speedup vs baseline: 1.4210x; 1.4210x over previous
import functools

import jax
import jax.numpy as jnp
from jax import lax
from jax.experimental import pallas as pl
from jax.experimental.pallas import tpu as pltpu
from jax.experimental.pallas import tpu_sc as plsc

E = 320000
D = 128
G = 256
NC = 2
NS = 16
L = 16
NW = NC * NS
EW = E // NW
C = 80
NCHUNK = EW // C
GROUPS = C // L


def _tanh(z):
    a = jnp.abs(z)
    e = jnp.exp(a * (-2.0))
    return jnp.sign(z) * (1.0 - e) / (1.0 + e)


def _sc_body(feats_hbm, ids_hbm, w_hbm, b_hbm, out_hbm,
             bufA, idsA, bufB, idsB, acc, wbuf, bbuf, smat, semA, semB):
    wid = lax.axis_index("s") * NC + lax.axis_index("c")
    row0 = wid * EW

    pltpu.sync_copy(w_hbm, wbuf)
    pltpu.sync_copy(b_hbm, bbuf)
    bv = bbuf[...]
    wv = [wbuf[pl.ds(j * L, L)] for j in range(D // L)]
    lanes = lax.iota(jnp.int32, L)

    def _zero(i, _):
        acc[pl.ds(i * L, L)] = jnp.zeros((L,), jnp.float32)
        return 0
    lax.fori_loop(0, G * D // L, _zero, 0)

    def _start(ci, buf, ids, sem):
        base = row0 + ci * C
        pltpu.async_copy(feats_hbm.at[pl.ds(base, C)], buf, sem)
        pltpu.async_copy(ids_hbm.at[pl.ds(base, C)], ids, sem)

    def _wait(buf, ids, sem):
        pltpu.make_async_copy(feats_hbm.at[pl.ds(0, C)], buf, sem).wait()
        pltpu.make_async_copy(ids_hbm.at[pl.ds(0, C)], ids, sem).wait()

    def _dg(x, perm):
        return x.at[perm].get(mode="promise_in_bounds")

    splat0 = jnp.zeros((L,), jnp.int32)
    splat1 = jnp.full((L,), 1, jnp.int32)
    p1, p2, p4, p8 = (lanes ^ 1, lanes ^ 2, lanes ^ 4, lanes ^ 8)
    m1 = (lanes & 1) == 0

    def _process(buf, ids_v):
        def _pair_accumulate(row_a, gsum):
            xa = [buf[row_a, pl.ds(j * L, L)] for j in range(D // L)]
            xb = [buf[row_a + 1, pl.ds(j * L, L)] for j in range(D // L)]
            sa = xa[0] * wv[0]
            sb = xb[0] * wv[0]
            for j in range(1, D // L):
                sa = sa + xa[j] * wv[j]
                sb = sb + xb[j] * wv[j]
            c = jnp.where(m1, sa, sb) + jnp.where(m1, _dg(sa, p1), _dg(sb, p1))
            t = c + _dg(c, p2)
            t = t + _dg(t, p4)
            t = t + _dg(t, p8)
            wp = _tanh(t + bv)
            wa = _dg(wp, splat0)
            wb = _dg(wp, splat1)
            return [gsum[j] + xa[j] * wa + xb[j] * wb
                    for j in range(D // L)]

        def _group(gi, _):
            r0 = gi * L
            idv = jnp.minimum(ids_v[pl.ds(r0, L)], G - 1)
            seg_first = idv[0]
            seg_last = idv[L - 1]

            def _uniform(_):
                zero = [jnp.zeros((L,), jnp.float32)] * (D // L)

                @plsc.parallel_loop(0, L, 2, unroll=2, carry=zero)
                def gsum(r, gs):
                    return _pair_accumulate(r0 + r, gs)

                off = seg_first * D
                for j in range(D // L):
                    plsc.addupdate(acc.at[pl.ds(off + j * L, L)], gsum[j])
                return 0

            def _mixed(_):
                for r in range(L):
                    row = r0 + r
                    s = buf[row, pl.ds(0, L)] * wv[0]
                    for j in range(1, D // L):
                        s = s + buf[row, pl.ds(j * L, L)] * wv[j]
                    smat[pl.ds(r * L, L)] = s
                hv = plsc.load_gather(smat, [lanes * L])
                for j in range(1, L):
                    hv = hv + plsc.load_gather(smat, [lanes * L + j])
                wg = _tanh(hv + bv)
                for r in range(L):
                    rfull = jnp.full((L,), r, jnp.int32)
                    wsp = _dg(wg, rfull)
                    base_idx = _dg(idv, rfull) * D + lanes
                    for j in range(D // L):
                        x = buf[r0 + r, pl.ds(j * L, L)] * wsp
                        plsc.addupdate_scatter(acc, [base_idx + j * L], x)
                return 0

            lax.cond(seg_first == seg_last, _uniform, _mixed, 0)
            return 0

        lax.fori_loop(0, GROUPS, _group, 0)

    _start(0, bufA, idsA, semA)

    def _pair(p, _):
        ci = p * 2
        _wait(bufA, idsA, semA)
        _start(ci + 1, bufB, idsB, semB)
        _process(bufA, idsA)
        _wait(bufB, idsB, semB)
        _start(ci + 2, bufA, idsA, semA)
        _process(bufB, idsB)
        return 0

    lax.fori_loop(0, NCHUNK // 2, _pair, 0)
    _wait(bufA, idsA, semA)
    _process(bufA, idsA)
    pltpu.sync_copy(acc, out_hbm.at[wid])


def _combine_body(parts_ref, o_ref):
    o_ref[...] = jnp.sum(parts_ref[...], axis=0)


@jax.jit
def _run(edge_feats, ids32, w_flat, b_pad):
    mesh = plsc.VectorSubcoreMesh(core_axis_name="c", subcore_axis_name="s",
                                  num_cores=NC, num_subcores=NS)
    sc = pl.kernel(
        _sc_body,
        out_type=jax.ShapeDtypeStruct((NW, G * D), jnp.float32),
        mesh=mesh,
        compiler_params=pltpu.CompilerParams(needs_layout_passes=False),
        scratch_types=[
            pltpu.VMEM((C, D), jnp.float32),
            pltpu.VMEM((C,), jnp.int32),
            pltpu.VMEM((C, D), jnp.float32),
            pltpu.VMEM((C,), jnp.int32),
            pltpu.VMEM((G * D,), jnp.float32),
            pltpu.VMEM((D,), jnp.float32),
            pltpu.VMEM((L,), jnp.float32),
            pltpu.VMEM((L * L,), jnp.float32),
            pltpu.SemaphoreType.DMA,
            pltpu.SemaphoreType.DMA,
        ],
    )
    parts = sc(edge_feats, ids32, w_flat, b_pad)
    out = pl.pallas_call(
        _combine_body,
        out_shape=jax.ShapeDtypeStruct((G, D), jnp.float32),
    )(parts.reshape(NW, G, D))
    return out


def kernel(edge_feats, segment_ids, num_graphs, W, b):
    ids32 = segment_ids.astype(jnp.int32)
    w_flat = W.reshape(D)
    b_pad = jnp.broadcast_to(b.reshape(1), (L,)).astype(jnp.float32)
    return _run(edge_feats, ids32, w_flat, b_pad)


# speedup vs baseline: 1.8280x; 1.2865x over previous
import functools

import jax
import jax.numpy as jnp
from jax import lax
from jax.experimental import pallas as pl
from jax.experimental.pallas import tpu as pltpu
from jax.experimental.pallas import tpu_sc as plsc

E = 320000
D = 128
G = 256
NC = 2
NS = 16
L = 16
NW = NC * NS
EW = E // NW
C = 80
NCHUNK = EW // C
GROUPS = C // L


def _tanh(z):
    a = jnp.abs(z)
    e = jnp.exp(a * (-2.0))
    return jnp.sign(z) * (1.0 - e) / (1.0 + e)


def _sc_body(feats_hbm, ids_hbm, w_hbm, b_hbm, out_hbm,
             bufA, idsA, bufB, idsB, acc, wbuf, bbuf, smat, semA, semB):
    wid = lax.axis_index("s") * NC + lax.axis_index("c")
    row0 = wid * EW

    pltpu.sync_copy(w_hbm, wbuf)
    pltpu.sync_copy(b_hbm, bbuf)
    bv = bbuf[...]
    wv = [wbuf[pl.ds(j * L, L)] for j in range(D // L)]
    lanes = lax.iota(jnp.int32, L)

    def _zero(i, _):
        acc[pl.ds(i * L, L)] = jnp.zeros((L,), jnp.float32)
        return 0
    lax.fori_loop(0, G * D // L, _zero, 0)

    def _start(ci, buf, ids, sem):
        base = row0 + ci * C
        pltpu.async_copy(feats_hbm.at[pl.ds(base, C)], buf, sem)
        pltpu.async_copy(ids_hbm.at[pl.ds(base, C)], ids, sem)

    def _wait(buf, ids, sem):
        pltpu.make_async_copy(feats_hbm.at[pl.ds(0, C)], buf, sem).wait()
        pltpu.make_async_copy(ids_hbm.at[pl.ds(0, C)], ids, sem).wait()

    def _dg(x, perm):
        return x.at[perm].get(mode="promise_in_bounds")

    splats = [jnp.full((L,), k, jnp.int32) for k in range(4)]
    p1, p2, p4, p8 = (lanes ^ 1, lanes ^ 2, lanes ^ 4, lanes ^ 8)
    m1 = (lanes & 1) == 0
    m2 = (lanes & 2) == 0

    def _process(buf, ids_v):
        def _quad_accumulate(row_a, gsum):
            xs = [[buf[row_a + k, pl.ds(j * L, L)] for j in range(D // L)]
                  for k in range(4)]
            ss = []
            for k in range(4):
                s = xs[k][0] * wv[0]
                for j in range(1, D // L):
                    s = s + xs[k][j] * wv[j]
                ss.append(s)

            def _halfred(sa, sb):
                c = (jnp.where(m1, sa, sb)
                     + jnp.where(m1, _dg(sa, p1), _dg(sb, p1)))
                t = c + _dg(c, p2)
                return t + _dg(t, p4)

            u = jnp.where(m2, _halfred(ss[0], ss[1]), _halfred(ss[2], ss[3]))
            u = u + _dg(u, p8)
            wq = _tanh(u + bv)
            ws = [_dg(wq, sp) for sp in splats]
            return [gsum[j] + ((xs[0][j] * ws[0] + xs[1][j] * ws[1])
                               + (xs[2][j] * ws[2] + xs[3][j] * ws[3]))
                    for j in range(D // L)]

        def _group(gi, _):
            r0 = gi * L
            idv = jnp.minimum(ids_v[pl.ds(r0, L)], G - 1)
            seg_first = idv[0]
            seg_last = idv[L - 1]

            def _uniform(_):
                gsum = [jnp.zeros((L,), jnp.float32)] * (D // L)
                for r in range(0, L, 4):
                    gsum = _quad_accumulate(r0 + r, gsum)
                off = seg_first * D
                for j in range(D // L):
                    plsc.addupdate(acc.at[pl.ds(off + j * L, L)], gsum[j])
                return 0

            def _mixed(_):
                for r in range(L):
                    row = r0 + r
                    s = buf[row, pl.ds(0, L)] * wv[0]
                    for j in range(1, D // L):
                        s = s + buf[row, pl.ds(j * L, L)] * wv[j]
                    smat[pl.ds(r * L, L)] = s
                hv = plsc.load_gather(smat, [lanes * L])
                for j in range(1, L):
                    hv = hv + plsc.load_gather(smat, [lanes * L + j])
                wg = _tanh(hv + bv)
                for r in range(L):
                    rfull = jnp.full((L,), r, jnp.int32)
                    wsp = _dg(wg, rfull)
                    base_idx = _dg(idv, rfull) * D + lanes
                    for j in range(D // L):
                        x = buf[r0 + r, pl.ds(j * L, L)] * wsp
                        plsc.addupdate_scatter(acc, [base_idx + j * L], x)
                return 0

            lax.cond(seg_first == seg_last, _uniform, _mixed, 0)
            return 0

        lax.fori_loop(0, GROUPS, _group, 0)

    _start(0, bufA, idsA, semA)

    def _pair(p, _):
        ci = p * 2
        _wait(bufA, idsA, semA)
        _start(ci + 1, bufB, idsB, semB)
        _process(bufA, idsA)
        _wait(bufB, idsB, semB)
        _start(ci + 2, bufA, idsA, semA)
        _process(bufB, idsB)
        return 0

    lax.fori_loop(0, NCHUNK // 2, _pair, 0)
    _wait(bufA, idsA, semA)
    _process(bufA, idsA)
    pltpu.sync_copy(acc, out_hbm.at[wid])


def _combine_body(parts_ref, o_ref):
    o_ref[...] = jnp.sum(parts_ref[...], axis=0)


@jax.jit
def _run(edge_feats, ids32, w_flat, b_pad):
    mesh = plsc.VectorSubcoreMesh(core_axis_name="c", subcore_axis_name="s",
                                  num_cores=NC, num_subcores=NS)
    sc = pl.kernel(
        _sc_body,
        out_type=jax.ShapeDtypeStruct((NW, G * D), jnp.float32),
        mesh=mesh,
        compiler_params=pltpu.CompilerParams(needs_layout_passes=False),
        scratch_types=[
            pltpu.VMEM((C, D), jnp.float32),
            pltpu.VMEM((C,), jnp.int32),
            pltpu.VMEM((C, D), jnp.float32),
            pltpu.VMEM((C,), jnp.int32),
            pltpu.VMEM((G * D,), jnp.float32),
            pltpu.VMEM((D,), jnp.float32),
            pltpu.VMEM((L,), jnp.float32),
            pltpu.VMEM((L * L,), jnp.float32),
            pltpu.SemaphoreType.DMA,
            pltpu.SemaphoreType.DMA,
        ],
    )
    parts = sc(edge_feats, ids32, w_flat, b_pad)
    out = pl.pallas_call(
        _combine_body,
        out_shape=jax.ShapeDtypeStruct((G, D), jnp.float32),
    )(parts.reshape(NW, G, D))
    return out


def kernel(edge_feats, segment_ids, num_graphs, W, b):
    ids32 = segment_ids.astype(jnp.int32)
    w_flat = W.reshape(D)
    b_pad = jnp.broadcast_to(b.reshape(1), (L,)).astype(jnp.float32)
    return _run(edge_feats, ids32, w_flat, b_pad)


# speedup vs baseline: 1.8444x; 1.0090x over previous
import functools

import jax
import jax.numpy as jnp
from jax import lax
from jax.experimental import pallas as pl
from jax.experimental.pallas import tpu as pltpu
from jax.experimental.pallas import tpu_sc as plsc

E = 320000
D = 128
G = 256
NC = 2
NS = 16
L = 16
NW = NC * NS
EW = E // NW
C = 80
NCHUNK = EW // C
GROUPS = C // L


def _tanh(z):
    a = jnp.abs(z)
    e = jnp.exp(a * (-2.0))
    return jnp.sign(z) * (1.0 - e) / (1.0 + e)


def _sc_body(feats_hbm, ids_hbm, w_hbm, b_hbm, out_hbm,
             bufA, idsA, bufB, idsB, acc, wbuf, bbuf, smat, semA, semB):
    wid = lax.axis_index("s") * NC + lax.axis_index("c")
    row0 = wid * EW

    pltpu.sync_copy(w_hbm, wbuf)
    pltpu.sync_copy(b_hbm, bbuf)
    bv = bbuf[...]
    wv = [wbuf[pl.ds(j * L, L)] for j in range(D // L)]
    lanes = lax.iota(jnp.int32, L)

    def _zero(i, _):
        acc[pl.ds(i * L, L)] = jnp.zeros((L,), jnp.float32)
        return 0
    lax.fori_loop(0, G * D // L, _zero, 0)

    def _start(ci, buf, ids, sem):
        base = row0 + ci * C
        pltpu.async_copy(feats_hbm.at[pl.ds(base, C)], buf, sem)
        pltpu.async_copy(ids_hbm.at[pl.ds(base, C)], ids, sem)

    def _wait(buf, ids, sem):
        pltpu.make_async_copy(feats_hbm.at[pl.ds(0, C)], buf, sem).wait()
        pltpu.make_async_copy(ids_hbm.at[pl.ds(0, C)], ids, sem).wait()

    def _dg(x, perm):
        return x.at[perm].get(mode="promise_in_bounds")

    splat0 = jnp.zeros((L,), jnp.int32)
    splat1 = jnp.full((L,), 1, jnp.int32)
    p1, p2, p4, p8 = (lanes ^ 1, lanes ^ 2, lanes ^ 4, lanes ^ 8)
    m1 = (lanes & 1) == 0

    def _process(buf, ids_v):
        def _pair_accumulate(row_a, gsum):
            xa = [buf[row_a, pl.ds(j * L, L)] for j in range(D // L)]
            xb = [buf[row_a + 1, pl.ds(j * L, L)] for j in range(D // L)]
            sa = xa[0] * wv[0]
            sb = xb[0] * wv[0]
            for j in range(1, D // L):
                sa = sa + xa[j] * wv[j]
                sb = sb + xb[j] * wv[j]
            c = jnp.where(m1, sa, sb) + jnp.where(m1, _dg(sa, p1), _dg(sb, p1))
            t = c + _dg(c, p2)
            t = t + _dg(t, p4)
            t = t + _dg(t, p8)
            wp = _tanh(t + bv)
            wa = _dg(wp, splat0)
            wb = _dg(wp, splat1)
            return [gsum[j] + xa[j] * wa + xb[j] * wb
                    for j in range(D // L)]

        def _group(gi, _):
            r0 = gi * L
            idv = jnp.minimum(ids_v[pl.ds(r0, L)], G - 1)
            seg_first = idv[0]
            seg_last = idv[L - 1]

            def _uniform(_):
                gsum = [jnp.zeros((L,), jnp.float32)] * (D // L)
                for r in range(0, L, 2):
                    gsum = _pair_accumulate(r0 + r, gsum)
                off = seg_first * D
                for j in range(D // L):
                    plsc.addupdate(acc.at[pl.ds(off + j * L, L)], gsum[j])
                return 0

            def _mixed(_):
                for r in range(L):
                    row = r0 + r
                    s = buf[row, pl.ds(0, L)] * wv[0]
                    for j in range(1, D // L):
                        s = s + buf[row, pl.ds(j * L, L)] * wv[j]
                    smat[pl.ds(r * L, L)] = s
                hv = plsc.load_gather(smat, [lanes * L])
                for j in range(1, L):
                    hv = hv + plsc.load_gather(smat, [lanes * L + j])
                wg = _tanh(hv + bv)
                for r in range(L):
                    rfull = jnp.full((L,), r, jnp.int32)
                    wsp = _dg(wg, rfull)
                    base_idx = _dg(idv, rfull) * D + lanes
                    for j in range(D // L):
                        x = buf[r0 + r, pl.ds(j * L, L)] * wsp
                        plsc.addupdate_scatter(acc, [base_idx + j * L], x)
                return 0

            lax.cond(seg_first == seg_last, _uniform, _mixed, 0)
            return 0

        lax.fori_loop(0, GROUPS, _group, 0)

    _start(0, bufA, idsA, semA)

    def _pair(p, _):
        ci = p * 2
        _wait(bufA, idsA, semA)
        _start(ci + 1, bufB, idsB, semB)
        _process(bufA, idsA)
        _wait(bufB, idsB, semB)
        _start(ci + 2, bufA, idsA, semA)
        _process(bufB, idsB)
        return 0

    lax.fori_loop(0, NCHUNK // 2, _pair, 0)
    _wait(bufA, idsA, semA)
    _process(bufA, idsA)
    pltpu.sync_copy(acc, out_hbm.at[wid])


def _combine_body(parts_ref, o_ref):
    o_ref[...] = jnp.sum(parts_ref[...], axis=0)


@jax.jit
def _run(edge_feats, ids32, w_flat, b_pad):
    mesh = plsc.VectorSubcoreMesh(core_axis_name="c", subcore_axis_name="s",
                                  num_cores=NC, num_subcores=NS)
    sc = pl.kernel(
        _sc_body,
        out_type=jax.ShapeDtypeStruct((NW, G * D), jnp.float32),
        mesh=mesh,
        compiler_params=pltpu.CompilerParams(needs_layout_passes=False),
        scratch_types=[
            pltpu.VMEM((C, D), jnp.float32),
            pltpu.VMEM((C,), jnp.int32),
            pltpu.VMEM((C, D), jnp.float32),
            pltpu.VMEM((C,), jnp.int32),
            pltpu.VMEM((G * D,), jnp.float32),
            pltpu.VMEM((D,), jnp.float32),
            pltpu.VMEM((L,), jnp.float32),
            pltpu.VMEM((L * L,), jnp.float32),
            pltpu.SemaphoreType.DMA,
            pltpu.SemaphoreType.DMA,
        ],
    )
    parts = sc(edge_feats, ids32, w_flat, b_pad)
    out = pl.pallas_call(
        _combine_body,
        out_shape=jax.ShapeDtypeStruct((G, D), jnp.float32),
    )(parts.reshape(NW, G, D))
    return out


def kernel(edge_feats, segment_ids, num_graphs, W, b):
    ids32 = segment_ids.astype(jnp.int32)
    w_flat = W.reshape(D)
    b_pad = jnp.broadcast_to(b.reshape(1), (L,)).astype(jnp.float32)
    return _run(edge_feats, ids32, w_flat, b_pad)
